# baseline (speedup 1.0000x reference)
import functools

import jax
import jax.numpy as jnp
from jax import lax
from jax.experimental import pallas as pl
from jax.experimental.pallas import tpu as pltpu
from jax.experimental.pallas import tpu_sc as plsc

NC = 2
NS = 16
L = 16

D = 128
NN = 32
G = 4
ROWS = G * NN
NBUF = 4

FAST_C = 0
F_SEEDS = 640
B_PAD = NS * F_SEEDS


def _gather_mean(x, nodes_p, neigh2):
    fg = F_SEEDS // G
    mesh = plsc.VectorSubcoreMesh(core_axis_name="c", subcore_axis_name="s")

    @functools.partial(
        pl.kernel,
        mesh=mesh,
        out_type=[
            jax.ShapeDtypeStruct((B_PAD, D), jnp.float32),
            jax.ShapeDtypeStruct((B_PAD, D), jnp.float32),
        ],
        scratch_types=[
            pltpu.VMEM((fg, ROWS), jnp.int32),
            pltpu.VMEM((F_SEEDS,), jnp.int32),
            pltpu.VMEM((128, D), jnp.float32),
            pltpu.VMEM((NBUF, ROWS, D), jnp.float32),
            pltpu.VMEM((NBUF, G, D), jnp.float32),
            pltpu.SemaphoreType.DMA((NBUF,)),
            pltpu.SemaphoreType.DMA((NBUF,)),
            pltpu.SemaphoreType.DMA,
        ],
    )
    def k(x_hbm, nodes_hbm, neigh_hbm, hs_hbm, ha_hbm,
          nidx_v, sidx_v, sbuf, nbuf, hbuf, gsem, osem, ssem):
        c = lax.axis_index("c")
        s = lax.axis_index("s")

        @pl.when(c == FAST_C)
        def _fast_core():
            base_row = pl.multiple_of(s * F_SEEDS, 64)
            gbase = pl.multiple_of(base_row // G, 8)
            pltpu.sync_copy(neigh_hbm.at[pl.ds(gbase, fg)], nidx_v)
            pltpu.sync_copy(nodes_hbm.at[pl.ds(base_row, F_SEEDS)], sidx_v)

            def self_loop(i, carry):
                lo = i * 128
                pltpu.async_copy(
                    x_hbm.at[sidx_v.at[pl.ds(lo, 128)]], sbuf, ssem
                ).wait()
                pltpu.sync_copy(sbuf, hs_hbm.at[pl.ds(base_row + lo, 128)])
                return carry

            lax.fori_loop(0, F_SEEDS // 128, self_loop, 0)

            def gather(g, slot):
                return pltpu.make_async_copy(
                    x_hbm.at[nidx_v.at[g]], nbuf.at[slot], gsem.at[slot]
                )

            def agg_write(g, slot):
                return pltpu.make_async_copy(
                    hbuf.at[slot],
                    ha_hbm.at[pl.ds(base_row + g * G, G)],
                    osem.at[slot],
                )

            for slot in range(NBUF):
                gather(slot, slot).start()

            n_outer = fg // NBUF

            def outer(go, carry):
                for slot in range(NBUF):
                    g = go * NBUF + slot
                    gather(g, slot).wait()
                    @pl.when(go > 0)
                    def _():
                        agg_write(g - NBUF, slot).wait()
                    for si in range(G):
                        UNR = 8

                        def body(t, accs):
                            row0 = si * NN + t * UNR
                            for u in range(UNR):
                                accs = tuple(
                                    accs[ci] + nbuf[slot, row0 + u, pl.ds(ci * L, L)]
                                    for ci in range(D // L)
                                )
                            return accs

                        accs = lax.fori_loop(
                            0, NN // UNR, body,
                            tuple(jnp.zeros((L,), jnp.float32) for _ in range(D // L)),
                        )
                        for ci in range(D // L):
                            hbuf[slot, si, pl.ds(ci * L, L)] = accs[ci]
                    agg_write(g, slot).start()
                    @pl.when(go < n_outer - 1)
                    def _():
                        gather(g + NBUF, slot).start()
                return carry

            lax.fori_loop(0, n_outer, outer, 0)
            for slot in range(NBUF):
                agg_write((n_outer - 1) * NBUF + slot, slot).wait()

    return k(x, nodes_p, neigh2)


def _mm_body(hs_ref, ha_ref, wt_ref, wb_ref, b_ref, o_ref):
    dims = (((1,), (0,)), ((), ()))
    o_ref[...] = (
        lax.dot_general(hs_ref[...], wt_ref[...], dims,
                        preferred_element_type=jnp.float32)
        + lax.dot_general(ha_ref[...], wb_ref[...], dims,
                          preferred_element_type=jnp.float32)
        + b_ref[...]
    )


def _linear(hs, ha, W_top, W_bot, b, n_out):
    blk = 1024
    grid = hs.shape[0] // blk
    return pl.pallas_call(
        _mm_body,
        grid=(grid,),
        in_specs=[
            pl.BlockSpec((blk, D), lambda i: (i, 0)),
            pl.BlockSpec((blk, D), lambda i: (i, 0)),
            pl.BlockSpec((D, D), lambda i: (0, 0)),
            pl.BlockSpec((D, D), lambda i: (0, 0)),
            pl.BlockSpec((1, D), lambda i: (0, 0)),
        ],
        out_specs=pl.BlockSpec((blk, D), lambda i: (i, 0)),
        out_shape=jax.ShapeDtypeStruct((n_out, D), jnp.float32),
    )(hs, ha, W_top, W_bot, b.reshape(1, D))


def kernel(x, nodes, neigh_idx, W, b):
    B, n_neigh = neigh_idx.shape
    assert n_neigh == NN and x.shape[1] == D
    pad = B_PAD - B
    nodes_p = jnp.concatenate([nodes, jnp.zeros((pad,), jnp.int32)])
    neigh_p = jnp.concatenate([neigh_idx, jnp.zeros((pad, NN), jnp.int32)], axis=0)
    neigh2 = neigh_p.reshape(B_PAD // G, G * NN)
    hs, ha = _gather_mean(x, nodes_p, neigh2)
    return _linear(hs, ha, W[:D], W[D:] * jnp.float32(1.0 / NN), b, B)


# speedup vs baseline: 1.4704x; 1.4704x over previous
import functools

import jax
import jax.numpy as jnp
from jax import lax
from jax.experimental import pallas as pl
from jax.experimental.pallas import tpu as pltpu
from jax.experimental.pallas import tpu_sc as plsc

NC = 2
NS = 16
L = 16

D = 128
NN = 32
G = 4
ROWS = G * NN
NBUF = 4

FAST_C = 0
F_SEEDS = 608
S_SEEDS = 32
B_PAD = NS * (F_SEEDS + S_SEEDS)


def _gather_mean(x, nodes_p, neigh2):
    fg = F_SEEDS // G
    sg = S_SEEDS // G
    mesh = plsc.VectorSubcoreMesh(core_axis_name="c", subcore_axis_name="s")

    @functools.partial(
        pl.kernel,
        mesh=mesh,
        out_type=[
            jax.ShapeDtypeStruct((B_PAD, D), jnp.float32),
            jax.ShapeDtypeStruct((B_PAD, D), jnp.float32),
        ],
        scratch_types=[
            pltpu.VMEM((fg, ROWS), jnp.int32),
            pltpu.VMEM((F_SEEDS,), jnp.int32),
            pltpu.VMEM((128, D), jnp.float32),
            pltpu.VMEM((NBUF, ROWS, D), jnp.float32),
            pltpu.VMEM((NBUF, G, D), jnp.float32),
            pltpu.SemaphoreType.DMA((NBUF,)),
            pltpu.SemaphoreType.DMA((NBUF,)),
            pltpu.SemaphoreType.DMA,
        ],
    )
    def k(x_hbm, nodes_hbm, neigh_hbm, hs_hbm, ha_hbm,
          nidx_v, sidx_v, sbuf, nbuf, hbuf, gsem, osem, ssem):
        c = lax.axis_index("c")
        s = lax.axis_index("s")
        on_fast = c == FAST_C
        my_seeds = jnp.where(on_fast, F_SEEDS, S_SEEDS)
        base_row = jnp.where(on_fast, s * F_SEEDS, NS * F_SEEDS + s * S_SEEDS)
        base_row = pl.multiple_of(base_row, 32)
        n_groups = jnp.where(on_fast, fg, sg)
        gbase = base_row // G
        gbase = pl.multiple_of(gbase, 8)

        @pl.when(on_fast)
        def _():
            pltpu.sync_copy(neigh_hbm.at[pl.ds(gbase, fg)], nidx_v)
            pltpu.sync_copy(nodes_hbm.at[pl.ds(base_row, F_SEEDS)], sidx_v)

        @pl.when(jnp.logical_not(on_fast))
        def _():
            pltpu.sync_copy(
                neigh_hbm.at[pl.ds(gbase, sg)], nidx_v.at[pl.ds(0, sg)]
            )
            pltpu.sync_copy(
                nodes_hbm.at[pl.ds(base_row, S_SEEDS)],
                sidx_v.at[pl.ds(0, S_SEEDS)],
            )

        def self_chunk(lo, sz):
            pltpu.async_copy(
                x_hbm.at[sidx_v.at[pl.ds(lo, sz)]], sbuf.at[pl.ds(0, sz)], ssem
            ).wait()
            pltpu.sync_copy(
                sbuf.at[pl.ds(0, sz)], hs_hbm.at[pl.ds(base_row + lo, sz)]
            )

        def self_loop(i, carry):
            self_chunk(i * 128, 128)
            return carry

        n_full = my_seeds // 128
        lax.fori_loop(0, n_full, self_loop, 0)
        rem_lo = n_full * 128

        @pl.when(on_fast)
        def _():
            self_chunk(rem_lo, F_SEEDS % 128)

        @pl.when(jnp.logical_not(on_fast))
        def _():
            self_chunk(rem_lo, S_SEEDS)

        def gather(g, slot):
            return pltpu.make_async_copy(
                x_hbm.at[nidx_v.at[g]], nbuf.at[slot], gsem.at[slot]
            )

        def agg_write(g, slot):
            return pltpu.make_async_copy(
                hbuf.at[slot],
                ha_hbm.at[pl.ds(base_row + g * G, G)],
                osem.at[slot],
            )

        for slot in range(NBUF):
            gather(slot, slot).start()

        n_outer = n_groups // NBUF

        def outer(go, carry):
            for slot in range(NBUF):
                g = go * NBUF + slot
                gather(g, slot).wait()
                @pl.when(go > 0)
                def _():
                    agg_write(g - NBUF, slot).wait()
                for si in range(G):
                    UNR = 8

                    def body(t, accs):
                        row0 = si * NN + t * UNR
                        for u in range(UNR):
                            accs = tuple(
                                accs[ci] + nbuf[slot, row0 + u, pl.ds(ci * L, L)]
                                for ci in range(D // L)
                            )
                        return accs

                    accs = lax.fori_loop(
                        0, NN // UNR, body,
                        tuple(jnp.zeros((L,), jnp.float32) for _ in range(D // L)),
                    )
                    for ci in range(D // L):
                        hbuf[slot, si, pl.ds(ci * L, L)] = accs[ci]
                agg_write(g, slot).start()
                @pl.when(go < n_outer - 1)
                def _():
                    gather(g + NBUF, slot).start()
            return carry

        lax.fori_loop(0, n_outer, outer, 0)
        for slot in range(NBUF):
            agg_write((n_outer - 1) * NBUF + slot, slot).wait()

    return k(x, nodes_p, neigh2)


def _mm_body(hs_ref, ha_ref, wt_ref, wb_ref, b_ref, o_ref):
    dims = (((1,), (0,)), ((), ()))
    o_ref[...] = (
        lax.dot_general(hs_ref[...], wt_ref[...], dims,
                        preferred_element_type=jnp.float32)
        + lax.dot_general(ha_ref[...], wb_ref[...], dims,
                          preferred_element_type=jnp.float32)
        + b_ref[...]
    )


def _linear(hs, ha, W_top, W_bot, b, n_out):
    blk = 1024
    grid = hs.shape[0] // blk
    return pl.pallas_call(
        _mm_body,
        grid=(grid,),
        in_specs=[
            pl.BlockSpec((blk, D), lambda i: (i, 0)),
            pl.BlockSpec((blk, D), lambda i: (i, 0)),
            pl.BlockSpec((D, D), lambda i: (0, 0)),
            pl.BlockSpec((D, D), lambda i: (0, 0)),
            pl.BlockSpec((1, D), lambda i: (0, 0)),
        ],
        out_specs=pl.BlockSpec((blk, D), lambda i: (i, 0)),
        out_shape=jax.ShapeDtypeStruct((n_out, D), jnp.float32),
    )(hs, ha, W_top, W_bot, b.reshape(1, D))


def kernel(x, nodes, neigh_idx, W, b):
    B, n_neigh = neigh_idx.shape
    assert n_neigh == NN and x.shape[1] == D
    pad = B_PAD - B
    nodes_p = jnp.concatenate([nodes, jnp.zeros((pad,), jnp.int32)])
    neigh_p = jnp.concatenate([neigh_idx, jnp.zeros((pad, NN), jnp.int32)], axis=0)
    neigh2 = neigh_p.reshape(B_PAD // G, G * NN)
    hs, ha = _gather_mean(x, nodes_p, neigh2)
    return _linear(hs, ha, W[:D], W[D:] * jnp.float32(1.0 / NN), b, B)
